# Initial kernel scaffold; baseline (speedup 1.0000x reference)
import jax
import jax.numpy as jnp
from jax.experimental import pallas as pl


def kernel(feat, edge_index, W_self_0, W_neigh_0, b_0, W_self_1, W_neigh_1, b_1, W_self_2, W_neigh_2, b_2):
    raise NotImplementedError("write your pallas kernel here")


# speedup vs baseline: 4.0535x; 4.0535x over previous
import functools

import jax
import jax.numpy as jnp
from jax import lax
from jax.experimental import pallas as pl
from jax.experimental.pallas import tpu as pltpu
from jax.experimental.pallas import tpu_sc as plsc

NC = 2
NS = 16
LN = 128


def _mm0_body(h_ref, ws_ref, wn_ref, b_ref, s_ref, z_ref):
    h = h_ref[...]
    s_ref[...] = jnp.dot(h, ws_ref[...], preferred_element_type=jnp.float32) + b_ref[...]
    z_ref[...] = jnp.dot(h, wn_ref[...], preferred_element_type=jnp.float32)


def _comb_mm_body(sp_ref, a_ref, d_ref, ws_ref, wn_ref, b_ref, s_ref, z_ref):
    agg = a_ref[0] + a_ref[1]
    dg = jnp.maximum(d_ref[0, :, 0:1] + d_ref[1, :, 0:1], 1.0)
    h = jnp.maximum(sp_ref[...] + agg / dg, 0.0)
    s_ref[...] = jnp.dot(h, ws_ref[...], preferred_element_type=jnp.float32) + b_ref[...]
    z_ref[...] = jnp.dot(h, wn_ref[...], preferred_element_type=jnp.float32)


def _final_body(sp_ref, a_ref, d_ref, o_ref):
    agg = a_ref[0] + a_ref[1]
    dg = jnp.maximum(d_ref[0, :, 0:1] + d_ref[1, :, 0:1], 1.0)
    o_ref[...] = sp_ref[...] + agg / dg


def _mm0(h, ws, wn, b, blk):
    n, d = h.shape
    return pl.pallas_call(
        _mm0_body,
        grid=(n // blk,),
        in_specs=[
            pl.BlockSpec((blk, d), lambda i: (i, 0)),
            pl.BlockSpec((d, d), lambda i: (0, 0)),
            pl.BlockSpec((d, d), lambda i: (0, 0)),
            pl.BlockSpec((1, d), lambda i: (0, 0)),
        ],
        out_specs=[
            pl.BlockSpec((blk, d), lambda i: (i, 0)),
            pl.BlockSpec((blk, d), lambda i: (i, 0)),
        ],
        out_shape=[jax.ShapeDtypeStruct((n, d), jnp.float32)] * 2,
    )(h, ws, wn, b)


def _comb_mm(s_prev, apart, degpart, ws, wn, b, blk):
    n, d = s_prev.shape
    n_pad = apart.shape[1]
    return pl.pallas_call(
        _comb_mm_body,
        grid=(n // blk,),
        in_specs=[
            pl.BlockSpec((blk, d), lambda i: (i, 0)),
            pl.BlockSpec((2, blk, d), lambda i: (0, i, 0)),
            pl.BlockSpec((2, blk, d), lambda i: (0, i, 0)),
            pl.BlockSpec((d, d), lambda i: (0, 0)),
            pl.BlockSpec((d, d), lambda i: (0, 0)),
            pl.BlockSpec((1, d), lambda i: (0, 0)),
        ],
        out_specs=[
            pl.BlockSpec((blk, d), lambda i: (i, 0)),
            pl.BlockSpec((blk, d), lambda i: (i, 0)),
        ],
        out_shape=[jax.ShapeDtypeStruct((n, d), jnp.float32)] * 2,
    )(s_prev, apart, degpart, ws, wn, b)


def _final(s_prev, apart, degpart, blk):
    n, d = s_prev.shape
    return pl.pallas_call(
        _final_body,
        grid=(n // blk,),
        in_specs=[
            pl.BlockSpec((blk, d), lambda i: (i, 0)),
            pl.BlockSpec((2, blk, d), lambda i: (0, i, 0)),
            pl.BlockSpec((2, blk, d), lambda i: (0, i, 0)),
        ],
        out_specs=pl.BlockSpec((blk, d), lambda i: (i, 0)),
        out_shape=jax.ShapeDtypeStruct((n, d), jnp.float32),
    )(s_prev, apart, degpart)


def _make_sc_scatter(n_pad, d, n_chunks):
    rows_per_sub = n_pad // NS
    mesh = plsc.VectorSubcoreMesh(
        core_axis_name="c", subcore_axis_name="s", num_cores=NC, num_subcores=NS
    )
    out_type = [jax.ShapeDtypeStruct((NC, n_pad, d), jnp.float32)]
    scratch = [
        pltpu.VMEM((n_chunks, LN), jnp.int32),
        pltpu.VMEM((n_chunks, LN), jnp.int32),
        pltpu.VMEM((LN, d), jnp.float32),
        pltpu.VMEM_SHARED((n_pad, d), jnp.float32),
        pltpu.SemaphoreType.DMA,
    ]

    def body(z_hbm, src_hbm, dst_hbm, zrow_hbm, a_out,
             src_v, dst_v, row_v, a_sp, sem):
        cid = lax.axis_index("c")
        sid = lax.axis_index("s")
        wid = cid * NS + sid
        base = sid * rows_per_sub
        pltpu.sync_copy(src_hbm.at[wid], src_v)
        pltpu.sync_copy(dst_hbm.at[wid], dst_v)
        pltpu.sync_copy(zrow_hbm, a_sp.at[pl.ds(base, rows_per_sub)])
        plsc.subcore_barrier()

        def step(j, carry):
            pltpu.async_copy(z_hbm.at[src_v.at[j]], row_v, sem).wait()
            pltpu.sync_copy(row_v, a_sp.at[dst_v.at[j]], add=True)
            return carry

        lax.fori_loop(0, n_chunks, step, 0)
        plsc.subcore_barrier()
        pltpu.sync_copy(a_sp.at[pl.ds(base, rows_per_sub)],
                        a_out.at[cid, pl.ds(base, rows_per_sub), :])

    return pl.kernel(body, out_type=out_type, mesh=mesh, scratch_types=scratch)


def _make_sc_deg(n_pad, d, n_chunks):
    rows_per_sub = n_pad // NS
    mesh = plsc.VectorSubcoreMesh(
        core_axis_name="c", subcore_axis_name="s", num_cores=NC, num_subcores=NS
    )
    out_type = [jax.ShapeDtypeStruct((NC, n_pad, d), jnp.float32)]
    scratch = [
        pltpu.VMEM((n_chunks, LN), jnp.int32),
        pltpu.VMEM((LN, d), jnp.float32),
        pltpu.VMEM_SHARED((n_pad, d), jnp.float32),
    ]

    def body(dst_hbm, zrow_hbm, ones_hbm, deg_out, dst_v, ones_v, deg_sp):
        cid = lax.axis_index("c")
        sid = lax.axis_index("s")
        wid = cid * NS + sid
        base = sid * rows_per_sub
        pltpu.sync_copy(dst_hbm.at[wid], dst_v)
        pltpu.sync_copy(ones_hbm, ones_v)
        pltpu.sync_copy(zrow_hbm, deg_sp.at[pl.ds(base, rows_per_sub)])
        plsc.subcore_barrier()

        def step(j, carry):
            pltpu.sync_copy(ones_v, deg_sp.at[dst_v.at[j]], add=True)
            return carry

        lax.fori_loop(0, n_chunks, step, 0)
        plsc.subcore_barrier()
        pltpu.sync_copy(deg_sp.at[pl.ds(base, rows_per_sub)],
                        deg_out.at[cid, pl.ds(base, rows_per_sub), :])

    return pl.kernel(body, out_type=out_type, mesh=mesh, scratch_types=scratch)


def kernel(feat, edge_index, W_self_0, W_neigh_0, b_0,
           W_self_1, W_neigh_1, b_1, W_self_2, W_neigh_2, b_2):
    n, d = feat.shape
    e = edge_index.shape[1]
    nw = NC * NS
    n_chunks = -(-e // (nw * LN))
    e_pad = nw * LN * n_chunks
    n_pad = -(-(n + 1) // (NS * 8)) * (NS * 8)

    src = edge_index[0].astype(jnp.int32)
    dst = edge_index[1].astype(jnp.int32)
    pad = e_pad - e
    src3 = jnp.concatenate([src, jnp.zeros((pad,), jnp.int32)]).reshape(nw, n_chunks, LN)
    dst3 = jnp.concatenate([dst, jnp.full((pad,), n, jnp.int32)]).reshape(nw, n_chunks, LN)

    rows_per_sub = n_pad // NS
    zrow = jnp.zeros((rows_per_sub, d), jnp.float32)
    onesr = jnp.ones((LN, d), jnp.float32)

    b0 = b_0.reshape(1, d)
    b1 = b_1.reshape(1, d)
    b2 = b_2.reshape(1, d)

    blk = 2000
    sc_deg = _make_sc_deg(n_pad, d, n_chunks)
    sc = _make_sc_scatter(n_pad, d, n_chunks)

    degp, = sc_deg(dst3, zrow, onesr)
    s0, z0 = _mm0(feat, W_self_0, W_neigh_0, b0, blk)
    a0, = sc(z0, src3, dst3, zrow)
    s1, z1 = _comb_mm(s0, a0, degp, W_self_1, W_neigh_1, b1, blk)
    a1, = sc(z1, src3, dst3, zrow)
    s2, z2 = _comb_mm(s1, a1, degp, W_self_2, W_neigh_2, b2, blk)
    a2, = sc(z2, src3, dst3, zrow)
    return _final(s2, a2, degp, blk)
